# baseline (speedup 1.0000x reference)
import functools

import jax
import jax.numpy as jnp
from jax import lax
from jax.experimental import pallas as pl
from jax.experimental.pallas import tpu as pltpu
from jax.experimental.pallas import tpu_sc as plsc

INPUT_DIM = 512
N_CLASSES = 128
MAX_DEPTH = 12
N_NODES = 2**MAX_DEPTH - 1
N_LEAVES = 2**MAX_DEPTH
BATCH = 16384

NC = 2
NS = 16
L = 16
NW = NC * NS
ROWS_PER_W = BATCH // NW
PASS_ROWS = 64
NPASS = ROWS_PER_W // PASS_ROWS
NGRP = PASS_ROWS // L


def _prep_body(sf_ref, lp_ref, nf_ref, table_ref):
    sf = sf_ref[...]
    nf_ref[...] = jnp.clip(jnp.floor(sf), 0, INPUT_DIM - 1).astype(jnp.int32)
    p = lp_ref[...]
    m = jnp.max(p, axis=1, keepdims=True)
    e = jnp.exp(p - m)
    table_ref[...] = e / jnp.sum(e, axis=1, keepdims=True)


def _prep(split_features, leaf_probabilities):
    return pl.pallas_call(
        _prep_body,
        out_shape=[
            jax.ShapeDtypeStruct((N_NODES,), jnp.int32),
            jax.ShapeDtypeStruct((N_LEAVES, N_CLASSES), jnp.float32),
        ],
    )(split_features, leaf_probabilities)


NXBUF = 3


def _traverse_body(
    x_ref, nf_ref, th_ref, table_ref, out_ref,
    nf_v, th_v, xbufs, gidxs, rowbufs, xsems, rsems, osems,
):
    wid = lax.axis_index("s") * NC + lax.axis_index("c")
    base = wid * ROWS_PER_W

    def fire_x(p):
        b = p % NXBUF
        pltpu.async_copy(
            x_ref.at[pl.ds(base + p * PASS_ROWS, PASS_ROWS), :],
            xbufs[b], xsems[b],
        )

    for p in range(NXBUF):
        fire_x(p)
    pltpu.sync_copy(nf_ref, nf_v)
    pltpu.sync_copy(th_ref, th_v)

    lane = lax.iota(jnp.int32, L)
    zero = jnp.zeros((L,), jnp.int32)
    lrows = [g * L + lane for g in range(NGRP)]

    for p in range(NPASS):
        b = p % NXBUF
        rb = p % 2
        pltpu.make_async_copy(
            x_ref.at[pl.ds(base, PASS_ROWS), :], xbufs[b], xsems[b]
        ).wait()
        idxs = [zero] * NGRP
        for _ in range(MAX_DEPTH):
            for g in range(NGRP):
                feat = plsc.load_gather(nf_v, [idxs[g]])
                thr = plsc.load_gather(th_v, [idxs[g]])
                xv = plsc.load_gather(xbufs[b], [lrows[g], feat])
                dec = jnp.where(xv > thr, 1, 0).astype(jnp.int32)
                idxs[g] = idxs[g] * 2 + 1 + dec
        if p >= 2:
            pltpu.make_async_copy(
                rowbufs[rb],
                out_ref.at[pl.ds(base, PASS_ROWS), :],
                osems[rb],
            ).wait()
        for g in range(NGRP):
            gidxs[rb][pl.ds(g * L, L)] = idxs[g] - N_NODES
        pltpu.async_copy(table_ref.at[gidxs[rb]], rowbufs[rb], rsems[rb])
        if p + NXBUF < NPASS:
            fire_x(p + NXBUF)
        if p >= 1:
            pltpu.make_async_copy(
                table_ref.at[gidxs[1 - rb]], rowbufs[1 - rb], rsems[1 - rb]
            ).wait()
            pltpu.async_copy(
                rowbufs[1 - rb],
                out_ref.at[pl.ds(base + (p - 1) * PASS_ROWS, PASS_ROWS), :],
                osems[1 - rb],
            )

    last_rb = (NPASS - 1) % 2
    pltpu.make_async_copy(
        table_ref.at[gidxs[last_rb]], rowbufs[last_rb], rsems[last_rb]
    ).wait()
    pltpu.sync_copy(
        rowbufs[last_rb],
        out_ref.at[pl.ds(base + (NPASS - 1) * PASS_ROWS, PASS_ROWS), :],
    )
    pltpu.make_async_copy(
        rowbufs[1 - last_rb],
        out_ref.at[pl.ds(base, PASS_ROWS), :],
        osems[1 - last_rb],
    ).wait()


@functools.partial(
    pl.kernel,
    out_type=jax.ShapeDtypeStruct((BATCH, N_CLASSES), jnp.float32),
    mesh=plsc.VectorSubcoreMesh(
        core_axis_name="c", subcore_axis_name="s", num_cores=NC,
        num_subcores=NS,
    ),
    scratch_types=[
        pltpu.VMEM((N_NODES,), jnp.int32),
        pltpu.VMEM((N_NODES,), jnp.float32),
        pltpu.VMEM((PASS_ROWS, INPUT_DIM), jnp.float32),
        pltpu.VMEM((PASS_ROWS, INPUT_DIM), jnp.float32),
        pltpu.VMEM((PASS_ROWS, INPUT_DIM), jnp.float32),
        pltpu.VMEM((PASS_ROWS,), jnp.int32),
        pltpu.VMEM((PASS_ROWS,), jnp.int32),
        pltpu.VMEM((PASS_ROWS, N_CLASSES), jnp.float32),
        pltpu.VMEM((PASS_ROWS, N_CLASSES), jnp.float32),
        pltpu.SemaphoreType.DMA,
        pltpu.SemaphoreType.DMA,
        pltpu.SemaphoreType.DMA,
        pltpu.SemaphoreType.DMA,
        pltpu.SemaphoreType.DMA,
        pltpu.SemaphoreType.DMA,
        pltpu.SemaphoreType.DMA,
    ],
    compiler_params=pltpu.CompilerParams(needs_layout_passes=False),
)
def _traverse(
    x, nf, th, table, out,
    nf_v, th_v, xbuf0, xbuf1, xbuf2, gidx0, gidx1, rowbuf0, rowbuf1,
    xsem0, xsem1, xsem2, rsem0, rsem1, osem0, osem1,
):
    _traverse_body(
        x, nf, th, table, out,
        nf_v, th_v, (xbuf0, xbuf1, xbuf2), (gidx0, gidx1),
        (rowbuf0, rowbuf1),
        (xsem0, xsem1, xsem2), (rsem0, rsem1), (osem0, osem1),
    )


def kernel(x, split_features, split_thresholds, leaf_probabilities):
    nf, table = _prep(split_features, leaf_probabilities)
    return _traverse(x, nf, split_thresholds, table)


# speedup vs baseline: 1.0361x; 1.0361x over previous
import functools

import jax
import jax.numpy as jnp
from jax import lax
from jax.experimental import pallas as pl
from jax.experimental.pallas import tpu as pltpu
from jax.experimental.pallas import tpu_sc as plsc

INPUT_DIM = 512
N_CLASSES = 128
MAX_DEPTH = 12
N_NODES = 2**MAX_DEPTH - 1
N_LEAVES = 2**MAX_DEPTH
BATCH = 16384

NC = 2
NS = 16
L = 16
NW = NC * NS
ROWS_PER_W = BATCH // NW
PASS_ROWS = 32
NPASS = ROWS_PER_W // PASS_ROWS
NGRP = PASS_ROWS // L
NBUF = 4


def _prep_body(sf_ref, lp_ref, nf_ref, table_ref):
    sf = sf_ref[...]
    nf_ref[...] = jnp.clip(jnp.floor(sf), 0, INPUT_DIM - 1).astype(jnp.int32)
    p = lp_ref[...]
    m = jnp.max(p, axis=1, keepdims=True)
    e = jnp.exp(p - m)
    table_ref[...] = e / jnp.sum(e, axis=1, keepdims=True)


def _prep(split_features, leaf_probabilities):
    return pl.pallas_call(
        _prep_body,
        out_shape=[
            jax.ShapeDtypeStruct((N_NODES,), jnp.int32),
            jax.ShapeDtypeStruct((N_LEAVES, N_CLASSES), jnp.float32),
        ],
    )(split_features, leaf_probabilities)


def _traverse_body(
    x_ref, nf_ref, th_ref, table_ref, out_ref,
    nf_v, th_v, xbufs, gidxs, rowbufs, xsems, rsems, osems,
):
    wid = lax.axis_index("s") * NC + lax.axis_index("c")
    base = wid * ROWS_PER_W

    def fire_x(b, p):
        pltpu.async_copy(
            x_ref.at[pl.ds(base + p * PASS_ROWS, PASS_ROWS), :],
            xbufs[b], xsems[b],
        )

    for b in range(NBUF):
        fire_x(b, b)
    pltpu.sync_copy(nf_ref, nf_v)
    pltpu.sync_copy(th_ref, th_v)

    lane = lax.iota(jnp.int32, L)
    zero = jnp.zeros((L,), jnp.int32)
    lrows = [g * L + lane for g in range(NGRP)]

    def run_pass(b, pp):
        p = pp * NBUF + b
        pltpu.make_async_copy(
            x_ref.at[pl.ds(base, PASS_ROWS), :], xbufs[b], xsems[b]
        ).wait()
        idxs = [zero] * NGRP
        for _ in range(MAX_DEPTH):
            for g in range(NGRP):
                feat = plsc.load_gather(nf_v, [idxs[g]])
                thr = plsc.load_gather(th_v, [idxs[g]])
                xv = plsc.load_gather(xbufs[b], [lrows[g], feat])
                dec = jnp.where(xv > thr, 1, 0).astype(jnp.int32)
                idxs[g] = idxs[g] * 2 + 1 + dec
        @pl.when(pp > 0)
        def _():
            pltpu.make_async_copy(
                rowbufs[b], out_ref.at[pl.ds(base, PASS_ROWS), :], osems[b]
            ).wait()
        for g in range(NGRP):
            gidxs[b][pl.ds(g * L, L)] = idxs[g] - N_NODES
        pltpu.async_copy(table_ref.at[gidxs[b]], rowbufs[b], rsems[b])
        @pl.when(pp < NPASS // NBUF - 1)
        def _():
            fire_x(b, p + NBUF)
        bprev = (b - 1) % NBUF

        def drain_prev():
            pltpu.make_async_copy(
                table_ref.at[gidxs[bprev]], rowbufs[bprev], rsems[bprev]
            ).wait()
            pltpu.async_copy(
                rowbufs[bprev],
                out_ref.at[pl.ds(base + (p - 1) * PASS_ROWS, PASS_ROWS), :],
                osems[bprev],
            )

        if b == 0:
            pl.when(pp > 0)(drain_prev)
        else:
            drain_prev()

    def body(pp, carry):
        for b in range(NBUF):
            run_pass(b, pp)
        return carry

    lax.fori_loop(0, NPASS // NBUF, body, 0)

    last_b = (NPASS - 1) % NBUF
    pltpu.make_async_copy(
        table_ref.at[gidxs[last_b]], rowbufs[last_b], rsems[last_b]
    ).wait()
    pltpu.sync_copy(
        rowbufs[last_b],
        out_ref.at[pl.ds(base + (NPASS - 1) * PASS_ROWS, PASS_ROWS), :],
    )
    for b in range(NBUF - 1):
        pltpu.make_async_copy(
            rowbufs[b], out_ref.at[pl.ds(base, PASS_ROWS), :], osems[b]
        ).wait()


@functools.partial(
    pl.kernel,
    out_type=jax.ShapeDtypeStruct((BATCH, N_CLASSES), jnp.float32),
    mesh=plsc.VectorSubcoreMesh(
        core_axis_name="c", subcore_axis_name="s", num_cores=NC,
        num_subcores=NS,
    ),
    scratch_types=(
        [
            pltpu.VMEM((N_NODES,), jnp.int32),
            pltpu.VMEM((N_NODES,), jnp.float32),
        ]
        + [pltpu.VMEM((PASS_ROWS, INPUT_DIM), jnp.float32)] * NBUF
        + [pltpu.VMEM((PASS_ROWS,), jnp.int32)] * NBUF
        + [pltpu.VMEM((PASS_ROWS, N_CLASSES), jnp.float32)] * NBUF
        + [pltpu.SemaphoreType.DMA] * (3 * NBUF)
    ),
    compiler_params=pltpu.CompilerParams(needs_layout_passes=False),
)
def _traverse(x, nf, th, table, out, *scratch):
    nf_v, th_v = scratch[0], scratch[1]
    xbufs = scratch[2:2 + NBUF]
    gidxs = scratch[2 + NBUF:2 + 2 * NBUF]
    rowbufs = scratch[2 + 2 * NBUF:2 + 3 * NBUF]
    sems = scratch[2 + 3 * NBUF:]
    xsems = sems[0:NBUF]
    rsems = sems[NBUF:2 * NBUF]
    osems = sems[2 * NBUF:3 * NBUF]
    _traverse_body(
        x, nf, th, table, out,
        nf_v, th_v, xbufs, gidxs, rowbufs, xsems, rsems, osems,
    )


def kernel(x, split_features, split_thresholds, leaf_probabilities):
    nf, table = _prep(split_features, leaf_probabilities)
    return _traverse(x, nf, split_thresholds, table)


# speedup vs baseline: 1.1135x; 1.0747x over previous
import functools

import jax
import jax.numpy as jnp
from jax import lax
from jax.experimental import pallas as pl
from jax.experimental.pallas import tpu as pltpu
from jax.experimental.pallas import tpu_sc as plsc

INPUT_DIM = 512
N_CLASSES = 128
MAX_DEPTH = 12
N_NODES = 2**MAX_DEPTH - 1
N_LEAVES = 2**MAX_DEPTH
BATCH = 16384

NC = 2
NS = 16
L = 16
NW = NC * NS
ROWS_PER_W = BATCH // NW
PASS_ROWS = 32
NPASS = ROWS_PER_W // PASS_ROWS
NGRP = PASS_ROWS // L
NBUF = 4


def _prep_body(sf_ref, lp_ref, nf_ref, table_ref):
    sf = sf_ref[...]
    nf_ref[...] = jnp.clip(jnp.floor(sf), 0, INPUT_DIM - 1).astype(jnp.int32)
    p = lp_ref[...]
    m = jnp.max(p, axis=1, keepdims=True)
    e = jnp.exp(p - m)
    table_ref[...] = e / jnp.sum(e, axis=1, keepdims=True)


def _prep(split_features, leaf_probabilities):
    return pl.pallas_call(
        _prep_body,
        out_shape=[
            jax.ShapeDtypeStruct((N_NODES,), jnp.int32),
            jax.ShapeDtypeStruct((N_LEAVES, N_CLASSES), jnp.float32),
        ],
    )(split_features, leaf_probabilities)


def _traverse_body(
    x_ref, nf_ref, th_ref, table_ref, out_ref,
    nf_v, th_v, table_sh, xbufs, gidxs, rowbufs, xsems, rsems, osems,
):
    sid = lax.axis_index("s")
    wid = sid * NC + lax.axis_index("c")
    base = wid * ROWS_PER_W

    def fire_x(b, p):
        pltpu.async_copy(
            x_ref.at[pl.ds(base + p * PASS_ROWS, PASS_ROWS), :],
            xbufs[b], xsems[b],
        )

    for b in range(NBUF):
        fire_x(b, b)
    shrows = N_LEAVES // NS
    sh_lo = sid * shrows
    pltpu.sync_copy(
        table_ref.at[pl.ds(sh_lo, shrows), :],
        table_sh.at[pl.ds(sh_lo, shrows), :],
    )
    pltpu.sync_copy(nf_ref, nf_v)
    pltpu.sync_copy(th_ref, th_v)
    plsc.subcore_barrier()

    lane = lax.iota(jnp.int32, L)
    zero = jnp.zeros((L,), jnp.int32)
    lrows = [g * L + lane for g in range(NGRP)]

    def run_pass(b, pp):
        p = pp * NBUF + b
        pltpu.make_async_copy(
            x_ref.at[pl.ds(base, PASS_ROWS), :], xbufs[b], xsems[b]
        ).wait()
        idxs = [zero] * NGRP
        for _ in range(MAX_DEPTH):
            for g in range(NGRP):
                feat = plsc.load_gather(nf_v, [idxs[g]])
                thr = plsc.load_gather(th_v, [idxs[g]])
                xv = plsc.load_gather(xbufs[b], [lrows[g], feat])
                dec = jnp.where(xv > thr, 1, 0).astype(jnp.int32)
                idxs[g] = idxs[g] * 2 + 1 + dec
        @pl.when(pp > 0)
        def _():
            pltpu.make_async_copy(
                rowbufs[b], out_ref.at[pl.ds(base, PASS_ROWS), :], osems[b]
            ).wait()
        for g in range(NGRP):
            gidxs[b][pl.ds(g * L, L)] = idxs[g] - N_NODES
        pltpu.async_copy(table_sh.at[gidxs[b]], rowbufs[b], rsems[b])
        @pl.when(pp < NPASS // NBUF - 1)
        def _():
            fire_x(b, p + NBUF)
        bprev = (b - 1) % NBUF

        def drain_prev():
            pltpu.make_async_copy(
                table_sh.at[gidxs[bprev]], rowbufs[bprev], rsems[bprev]
            ).wait()
            pltpu.async_copy(
                rowbufs[bprev],
                out_ref.at[pl.ds(base + (p - 1) * PASS_ROWS, PASS_ROWS), :],
                osems[bprev],
            )

        if b == 0:
            pl.when(pp > 0)(drain_prev)
        else:
            drain_prev()

    def body(pp, carry):
        for b in range(NBUF):
            run_pass(b, pp)
        return carry

    lax.fori_loop(0, NPASS // NBUF, body, 0)

    last_b = (NPASS - 1) % NBUF
    pltpu.make_async_copy(
        table_sh.at[gidxs[last_b]], rowbufs[last_b], rsems[last_b]
    ).wait()
    pltpu.sync_copy(
        rowbufs[last_b],
        out_ref.at[pl.ds(base + (NPASS - 1) * PASS_ROWS, PASS_ROWS), :],
    )
    for b in range(NBUF - 1):
        pltpu.make_async_copy(
            rowbufs[b], out_ref.at[pl.ds(base, PASS_ROWS), :], osems[b]
        ).wait()


@functools.partial(
    pl.kernel,
    out_type=jax.ShapeDtypeStruct((BATCH, N_CLASSES), jnp.float32),
    mesh=plsc.VectorSubcoreMesh(
        core_axis_name="c", subcore_axis_name="s", num_cores=NC,
        num_subcores=NS,
    ),
    scratch_types=(
        [
            pltpu.VMEM((N_NODES,), jnp.int32),
            pltpu.VMEM((N_NODES,), jnp.float32),
            pltpu.VMEM_SHARED((N_LEAVES, N_CLASSES), jnp.float32),
        ]
        + [pltpu.VMEM((PASS_ROWS, INPUT_DIM), jnp.float32)] * NBUF
        + [pltpu.VMEM((PASS_ROWS,), jnp.int32)] * NBUF
        + [pltpu.VMEM((PASS_ROWS, N_CLASSES), jnp.float32)] * NBUF
        + [pltpu.SemaphoreType.DMA] * (3 * NBUF)
    ),
    compiler_params=pltpu.CompilerParams(needs_layout_passes=False),
)
def _traverse(x, nf, th, table, out, *scratch):
    nf_v, th_v, table_sh = scratch[0], scratch[1], scratch[2]
    xbufs = scratch[3:3 + NBUF]
    gidxs = scratch[3 + NBUF:3 + 2 * NBUF]
    rowbufs = scratch[3 + 2 * NBUF:3 + 3 * NBUF]
    sems = scratch[3 + 3 * NBUF:]
    xsems = sems[0:NBUF]
    rsems = sems[NBUF:2 * NBUF]
    osems = sems[2 * NBUF:3 * NBUF]
    _traverse_body(
        x, nf, th, table, out,
        nf_v, th_v, table_sh, xbufs, gidxs, rowbufs, xsems, rsems, osems,
    )


def kernel(x, split_features, split_thresholds, leaf_probabilities):
    nf, table = _prep(split_features, leaf_probabilities)
    return _traverse(x, nf, split_thresholds, table)
